# baseline (speedup 1.0000x reference)
import jax
import jax.numpy as jnp
from jax.experimental import pallas as pl
from jax.experimental.pallas import tpu as pltpu

_NEG_SLOPE = 0.2
_EPS = 1e-5


def _make_fused_kernel(H, W, C):
    H2, W2 = 2 * H, 2 * W

    def _conv9(src_ref, w_ref):
        t = jnp.concatenate([src_ref[:, 0:W2, :],
                             src_ref[:, 1:W2 + 1, :],
                             src_ref[:, 2:W2 + 2, :]], axis=-1)
        t2 = jnp.concatenate([t[0:H2], t[1:H2 + 1], t[2:H2 + 2]], axis=-1)
        acc = jnp.dot(t2.reshape(H2 * W2, 9 * C), w_ref[...],
                      preferred_element_type=jnp.float32)
        return acc.reshape(H2, W2, C)

    def _epilogue_ph(planes, nz_ref, nw_ref, sc_ref, sh_ref):
        nw = nw_ref[...].reshape(1, 1, C)
        yy = [[None, None], [None, None]]
        s1 = jnp.zeros((1, 1, C), jnp.float32)
        s2 = jnp.zeros((1, 1, C), jnp.float32)
        for r in range(2):
            for s in range(2):
                y = planes[r][s] + nw * nz_ref[0, r, s]
                y = jnp.where(y >= 0.0, y, _NEG_SLOPE * y)
                yy[r][s] = y
                s1 = s1 + jnp.sum(y, axis=(0, 1), keepdims=True)
                s2 = s2 + jnp.sum(jnp.square(y), axis=(0, 1), keepdims=True)
        inv = 1.0 / (H2 * W2)
        mean = s1 * inv
        rstd = jax.lax.rsqrt(s2 * inv - jnp.square(mean) + _EPS)
        a = rstd * sc_ref[...].reshape(1, 1, C)
        bb = sh_ref[...].reshape(1, 1, C) - mean * a
        return [[yy[r][s] * a + bb for s in range(2)] for r in range(2)]

    def _epilogue(y, nz_ref, nw_ref, sc_ref, sh_ref):
        y = y + nw_ref[...].reshape(1, 1, C) * nz_ref[0]
        y = jnp.where(y >= 0.0, y, _NEG_SLOPE * y)
        inv = 1.0 / (H2 * W2)
        mean = jnp.sum(y, axis=(0, 1), keepdims=True) * inv
        ex2 = jnp.sum(jnp.square(y), axis=(0, 1), keepdims=True) * inv
        rstd = jax.lax.rsqrt(ex2 - jnp.square(mean) + _EPS)
        a = rstd * sc_ref[...].reshape(1, 1, C)
        b = sh_ref[...].reshape(1, 1, C) - mean * a
        return y * a + b

    def _body(x_ref, w0_ref, b0_ref, nz1_ref, nw1_ref, sc1_ref, sh1_ref,
              w1_ref, b1_ref, nz2_ref, nw2_ref, sc2_ref, sh2_ref,
              o_ref, xp_ref, up_ref):
        xp_ref[0:1] = jnp.zeros((1, W + 2, C), jnp.bfloat16)
        xp_ref[H + 1:H + 2] = jnp.zeros((1, W + 2, C), jnp.bfloat16)
        xp_ref[:, 0:1] = jnp.zeros((H + 2, 1, C), jnp.bfloat16)
        xp_ref[:, W + 1:W + 2] = jnp.zeros((H + 2, 1, C), jnp.bfloat16)
        up_ref[0:1] = jnp.zeros((1, W2 + 2, C), jnp.bfloat16)
        up_ref[H2 + 1:H2 + 2] = jnp.zeros((1, W2 + 2, C), jnp.bfloat16)
        up_ref[:, 0:1] = jnp.zeros((H2 + 2, 1, C), jnp.bfloat16)
        up_ref[:, W2 + 1:W2 + 2] = jnp.zeros((H2 + 2, 1, C), jnp.bfloat16)

        xp_ref[1:H + 1, 1:W + 1, :] = x_ref[0].astype(jnp.bfloat16)
        c0 = xp_ref[:, 0:W, :]
        c1 = xp_ref[:, 1:W + 1, :]
        c2 = xp_ref[:, 2:W + 2, :]
        t_s = (jnp.concatenate([c0, c1], axis=-1),
               jnp.concatenate([c1, c2], axis=-1))
        ph = [[None, None], [None, None]]
        for r in range(2):
            for s in range(2):
                tap = jnp.concatenate([t_s[s][r:r + H],
                                       t_s[s][r + 1:r + H + 1]], axis=-1)
                acc = jnp.dot(tap.reshape(H * W, 4 * C), w0_ref[r, s],
                              preferred_element_type=jnp.float32)
                ph[r][s] = acc.reshape(H, W, C)

        zrow = jnp.zeros((1, W, C), jnp.float32)
        v = [[None, None], [None, None]]
        for s in range(2):
            up1 = jnp.concatenate([zrow, ph[1][s][0:H - 1]], axis=0)
            dn0 = jnp.concatenate([ph[0][s][1:H], zrow], axis=0)
            v[0][s] = 0.5 * ph[0][s] + 0.25 * (up1 + ph[1][s])
            v[1][s] = 0.5 * ph[1][s] + 0.25 * (ph[0][s] + dn0)
        zcol = jnp.zeros((H, 1, C), jnp.float32)
        b0v = b0_ref[...].reshape(1, 1, C)
        b = [[None, None], [None, None]]
        for r in range(2):
            lf1 = jnp.concatenate([zcol, v[r][1][:, 0:W - 1]], axis=1)
            rt0 = jnp.concatenate([v[r][0][:, 1:W], zcol], axis=1)
            b[r][0] = 0.5 * v[r][0] + 0.25 * (lf1 + v[r][1]) + b0v
            b[r][1] = 0.5 * v[r][1] + 0.25 * (v[r][0] + rt0) + b0v

        z = _epilogue_ph(b, nz1_ref, nw1_ref, sc1_ref, sh1_ref)
        zb = [[z[r][s].astype(jnp.bfloat16) for s in range(2)] for r in range(2)]
        y0 = jnp.stack([zb[0][0], zb[1][0]], axis=1).reshape(H2, W, C)
        y1 = jnp.stack([zb[0][1], zb[1][1]], axis=1).reshape(H2, W, C)
        y = jnp.stack([y0, y1], axis=2).reshape(H2, W2, C)

        up_ref[1:H2 + 1, 1:W2 + 1, :] = y
        y = _conv9(up_ref, w1_ref) + b1_ref[...].reshape(1, 1, C)

        y = _epilogue(y, nz2_ref, nw2_ref, sc2_ref, sh2_ref)
        o_ref[0] = y

    return _body


def _phase_split(noise, N, H, W):
    return jnp.transpose(noise.reshape(N, H, 2, W, 2),
                         (0, 2, 4, 1, 3)).reshape(N, 2, 2, H, W, 1)


def _style_affine(latent, w, b, w_mul, C):
    style = jnp.matmul(latent, (w * w_mul).T,
                       precision=jax.lax.Precision.HIGHEST) + b
    return style[:, :C] + 1.0, style[:, C:]


@jax.jit
def _forward(x_nchw, dlatents, params):
    N, Ci, H, W = x_nchw.shape
    Co = params["w0"].shape[0]
    H2, W2 = 2 * H, 2 * W

    x = jnp.transpose(x_nchw, (0, 2, 3, 1))

    w0f = jnp.transpose(params["w0"], (2, 3, 1, 0)) * params["w0_mul"]
    wr = (jnp.stack([w0f[0], w0f[1] + w0f[2]]),
          jnp.stack([w0f[0] + w0f[1], w0f[2]]))
    wp = [[jnp.stack([a[:, 0], a[:, 1] + a[:, 2]], axis=1),
           jnp.stack([a[:, 0] + a[:, 1], a[:, 2]], axis=1)]
          for a in wr]
    w0 = jnp.stack([jnp.stack(wp[0]), jnp.stack(wp[1])])
    w0 = w0.reshape(2, 2, 4 * Ci, Co).astype(jnp.bfloat16)
    w1 = (jnp.transpose(params["w1"], (2, 3, 1, 0)) * params["w1_mul"]).astype(jnp.bfloat16)
    w1 = w1.reshape(9 * Co, Co)
    b0 = params["b0"].reshape(1, Co) * params["b_mul"]
    b1 = params["b1"].reshape(1, Co) * params["b_mul"]
    sc1, sh1 = _style_affine(dlatents[:, 0], params["s1_w"], params["s1_b"],
                             params["s_mul"], Co)
    sc2, sh2 = _style_affine(dlatents[:, 1], params["s2_w"], params["s2_b"],
                             params["s_mul"], Co)
    nz1 = _phase_split(params["noise1"], N, H, W)
    nz2 = params["noise2"].reshape(N, H2, W2, 1)

    y = pl.pallas_call(
        _make_fused_kernel(H, W, Co),
        out_shape=jax.ShapeDtypeStruct((N, H2, W2, Co), x.dtype),
        grid_spec=pltpu.PrefetchScalarGridSpec(
            num_scalar_prefetch=0,
            grid=(N,),
            in_specs=[
                pl.BlockSpec((1, H, W, Ci), lambda n: (n, 0, 0, 0)),
                pl.BlockSpec((2, 2, 4 * Ci, Co), lambda n: (0, 0, 0, 0)),
                pl.BlockSpec((1, Co), lambda n: (0, 0)),
                pl.BlockSpec((1, 2, 2, H, W, 1), lambda n: (n, 0, 0, 0, 0, 0)),
                pl.BlockSpec((1, 1, Co), lambda n: (0, 0, 0)),
                pl.BlockSpec((1, 1, Co), lambda n: (n, 0, 0)),
                pl.BlockSpec((1, 1, Co), lambda n: (n, 0, 0)),
                pl.BlockSpec((9 * Co, Co), lambda n: (0, 0)),
                pl.BlockSpec((1, Co), lambda n: (0, 0)),
                pl.BlockSpec((1, H2, W2, 1), lambda n: (n, 0, 0, 0)),
                pl.BlockSpec((1, 1, Co), lambda n: (0, 0, 0)),
                pl.BlockSpec((1, 1, Co), lambda n: (n, 0, 0)),
                pl.BlockSpec((1, 1, Co), lambda n: (n, 0, 0)),
            ],
            out_specs=pl.BlockSpec((1, H2, W2, Co), lambda n: (n, 0, 0, 0)),
            scratch_shapes=[
                pltpu.VMEM((H + 2, W + 2, Ci), jnp.bfloat16),
                pltpu.VMEM((H2 + 2, W2 + 2, Co), jnp.bfloat16),
            ],
        ),
        compiler_params=pltpu.CompilerParams(dimension_semantics=("parallel",)),
    )(x, w0, b0, nz1, params["nw1"].reshape(1, 1, Co),
      sc1.reshape(N, 1, Co), sh1.reshape(N, 1, Co),
      w1, b1, nz2, params["nw2"].reshape(1, 1, Co),
      sc2.reshape(N, 1, Co), sh2.reshape(N, 1, Co))

    return jnp.transpose(y, (0, 3, 1, 2))


def kernel(x_nchw, dlatents, w0, w0_mul, b0, w1, w1_mul, b1, b_mul,
           nw1, nw2, noise1, noise2, s1_w, s1_b, s2_w, s2_b, s_mul):
    params = {"w0": w0, "w0_mul": w0_mul, "b0": b0, "w1": w1, "w1_mul": w1_mul,
              "b1": b1, "b_mul": b_mul, "nw1": nw1, "nw2": nw2,
              "noise1": noise1, "noise2": noise2, "s1_w": s1_w, "s1_b": s1_b,
              "s2_w": s2_w, "s2_b": s2_b, "s_mul": s_mul}
    return _forward(x_nchw, dlatents, params)


# speedup vs baseline: 1.0459x; 1.0459x over previous
import jax
import jax.numpy as jnp
from jax.experimental import pallas as pl
from jax.experimental.pallas import tpu as pltpu

_NEG_SLOPE = 0.2
_EPS = 1e-5


def _make_fused_kernel(H, W, C):
    H2, W2 = 2 * H, 2 * W

    def _conv9(src_ref, w_ref):
        t = jnp.concatenate([src_ref[:, 0:W2, :],
                             src_ref[:, 1:W2 + 1, :],
                             src_ref[:, 2:W2 + 2, :]], axis=-1)
        t2 = jnp.concatenate([t[0:H2], t[1:H2 + 1], t[2:H2 + 2]], axis=-1)
        acc = jnp.dot(t2.reshape(H2 * W2, 9 * C), w_ref[...],
                      preferred_element_type=jnp.float32)
        return acc.reshape(H2, W2, C)

    def _epilogue(y, nz_ref, nw_ref, sc_ref, sh_ref):
        y = y + nw_ref[...].reshape(1, 1, C) * nz_ref[0]
        y = jnp.where(y >= 0.0, y, _NEG_SLOPE * y)
        inv = 1.0 / (H2 * W2)
        mean = jnp.sum(y, axis=(0, 1), keepdims=True) * inv
        ex2 = jnp.sum(jnp.square(y), axis=(0, 1), keepdims=True) * inv
        rstd = jax.lax.rsqrt(ex2 - jnp.square(mean) + _EPS)
        a = rstd * sc_ref[...].reshape(1, 1, C)
        b = sh_ref[...].reshape(1, 1, C) - mean * a
        return y * a + b

    def _body(x_ref, w0_ref, b0_ref, nz1_ref, nw1_ref, sc1_ref, sh1_ref,
              w1_ref, b1_ref, nz2_ref, nw2_ref, sc2_ref, sh2_ref,
              o_ref, xp_ref, up_ref):
        xp_ref[0:1] = jnp.zeros((1, W + 2, C), jnp.bfloat16)
        xp_ref[H + 1:H + 2] = jnp.zeros((1, W + 2, C), jnp.bfloat16)
        xp_ref[:, 0:1] = jnp.zeros((H + 2, 1, C), jnp.bfloat16)
        xp_ref[:, W + 1:W + 2] = jnp.zeros((H + 2, 1, C), jnp.bfloat16)
        up_ref[0:1] = jnp.zeros((1, W2 + 2, C), jnp.bfloat16)
        up_ref[H2 + 1:H2 + 2] = jnp.zeros((1, W2 + 2, C), jnp.bfloat16)
        up_ref[:, 0:1] = jnp.zeros((H2 + 2, 1, C), jnp.bfloat16)
        up_ref[:, W2 + 1:W2 + 2] = jnp.zeros((H2 + 2, 1, C), jnp.bfloat16)

        xp_ref[1:H + 1, 1:W + 1, :] = x_ref[0].astype(jnp.bfloat16)
        c0 = xp_ref[:, 0:W, :]
        c1 = xp_ref[:, 1:W + 1, :]
        c2 = xp_ref[:, 2:W + 2, :]
        t_s = (jnp.concatenate([c0, c1], axis=-1),
               jnp.concatenate([c1, c2], axis=-1))
        ph = [[None, None], [None, None]]
        for r in range(2):
            for s in range(2):
                tap = jnp.concatenate([t_s[s][r:r + H],
                                       t_s[s][r + 1:r + H + 1]], axis=-1)
                acc = jnp.dot(tap.reshape(H * W, 4 * C), w0_ref[r, s],
                              preferred_element_type=jnp.float32)
                ph[r][s] = acc.reshape(H, W, C)

        zrow = jnp.zeros((1, W, C), jnp.float32)
        v = [[None, None], [None, None]]
        for s in range(2):
            up1 = jnp.concatenate([zrow, ph[1][s][0:H - 1]], axis=0)
            dn0 = jnp.concatenate([ph[0][s][1:H], zrow], axis=0)
            v[0][s] = 0.5 * ph[0][s] + 0.25 * (up1 + ph[1][s])
            v[1][s] = 0.5 * ph[1][s] + 0.25 * (ph[0][s] + dn0)
        zcol = jnp.zeros((H, 1, C), jnp.float32)
        b = [[None, None], [None, None]]
        for r in range(2):
            lf1 = jnp.concatenate([zcol, v[r][1][:, 0:W - 1]], axis=1)
            rt0 = jnp.concatenate([v[r][0][:, 1:W], zcol], axis=1)
            b[r][0] = 0.5 * v[r][0] + 0.25 * (lf1 + v[r][1])
            b[r][1] = 0.5 * v[r][1] + 0.25 * (v[r][0] + rt0)

        y0 = jnp.stack([b[0][0], b[1][0]], axis=1).reshape(H2, W, C)
        y1 = jnp.stack([b[0][1], b[1][1]], axis=1).reshape(H2, W, C)
        y = jnp.stack([y0, y1], axis=2).reshape(H2, W2, C)
        y = y + b0_ref[...].reshape(1, 1, C)

        y = _epilogue(y, nz1_ref, nw1_ref, sc1_ref, sh1_ref)

        up_ref[1:H2 + 1, 1:W2 + 1, :] = y.astype(jnp.bfloat16)
        y = _conv9(up_ref, w1_ref) + b1_ref[...].reshape(1, 1, C)

        y = _epilogue(y, nz2_ref, nw2_ref, sc2_ref, sh2_ref)
        o_ref[0] = y

    return _body


def _style_affine(latent, w, b, w_mul, C):
    style = jnp.matmul(latent, (w * w_mul).T,
                       precision=jax.lax.Precision.HIGHEST) + b
    return style[:, :C] + 1.0, style[:, C:]


@jax.jit
def _forward(x_nchw, dlatents, params):
    N, Ci, H, W = x_nchw.shape
    Co = params["w0"].shape[0]
    H2, W2 = 2 * H, 2 * W

    x = jnp.transpose(x_nchw, (0, 2, 3, 1))

    w0f = jnp.transpose(params["w0"], (2, 3, 1, 0)) * params["w0_mul"]
    wr = (jnp.stack([w0f[0], w0f[1] + w0f[2]]),
          jnp.stack([w0f[0] + w0f[1], w0f[2]]))
    wp = [[jnp.stack([a[:, 0], a[:, 1] + a[:, 2]], axis=1),
           jnp.stack([a[:, 0] + a[:, 1], a[:, 2]], axis=1)]
          for a in wr]
    w0 = jnp.stack([jnp.stack(wp[0]), jnp.stack(wp[1])])
    w0 = w0.reshape(2, 2, 4 * Ci, Co).astype(jnp.bfloat16)
    w1 = (jnp.transpose(params["w1"], (2, 3, 1, 0)) * params["w1_mul"]).astype(jnp.bfloat16)
    w1 = w1.reshape(9 * Co, Co)
    b0 = params["b0"].reshape(1, Co) * params["b_mul"]
    b1 = params["b1"].reshape(1, Co) * params["b_mul"]
    sc1, sh1 = _style_affine(dlatents[:, 0], params["s1_w"], params["s1_b"],
                             params["s_mul"], Co)
    sc2, sh2 = _style_affine(dlatents[:, 1], params["s2_w"], params["s2_b"],
                             params["s_mul"], Co)
    nz1 = params["noise1"].reshape(N, H2, W2, 1)
    nz2 = params["noise2"].reshape(N, H2, W2, 1)

    y = pl.pallas_call(
        _make_fused_kernel(H, W, Co),
        out_shape=jax.ShapeDtypeStruct((N, H2, W2, Co), x.dtype),
        grid_spec=pltpu.PrefetchScalarGridSpec(
            num_scalar_prefetch=0,
            grid=(N,),
            in_specs=[
                pl.BlockSpec((1, H, W, Ci), lambda n: (n, 0, 0, 0)),
                pl.BlockSpec((2, 2, 4 * Ci, Co), lambda n: (0, 0, 0, 0)),
                pl.BlockSpec((1, Co), lambda n: (0, 0)),
                pl.BlockSpec((1, H2, W2, 1), lambda n: (n, 0, 0, 0)),
                pl.BlockSpec((1, 1, Co), lambda n: (0, 0, 0)),
                pl.BlockSpec((1, 1, Co), lambda n: (n, 0, 0)),
                pl.BlockSpec((1, 1, Co), lambda n: (n, 0, 0)),
                pl.BlockSpec((9 * Co, Co), lambda n: (0, 0)),
                pl.BlockSpec((1, Co), lambda n: (0, 0)),
                pl.BlockSpec((1, H2, W2, 1), lambda n: (n, 0, 0, 0)),
                pl.BlockSpec((1, 1, Co), lambda n: (0, 0, 0)),
                pl.BlockSpec((1, 1, Co), lambda n: (n, 0, 0)),
                pl.BlockSpec((1, 1, Co), lambda n: (n, 0, 0)),
            ],
            out_specs=pl.BlockSpec((1, H2, W2, Co), lambda n: (n, 0, 0, 0)),
            scratch_shapes=[
                pltpu.VMEM((H + 2, W + 2, Ci), jnp.bfloat16),
                pltpu.VMEM((H2 + 2, W2 + 2, Co), jnp.bfloat16),
            ],
        ),
        compiler_params=pltpu.CompilerParams(dimension_semantics=("parallel",)),
    )(x, w0, b0, nz1, params["nw1"].reshape(1, 1, Co),
      sc1.reshape(N, 1, Co), sh1.reshape(N, 1, Co),
      w1, b1, nz2, params["nw2"].reshape(1, 1, Co),
      sc2.reshape(N, 1, Co), sh2.reshape(N, 1, Co))

    return jnp.transpose(y, (0, 3, 1, 2))


def kernel(x_nchw, dlatents, w0, w0_mul, b0, w1, w1_mul, b1, b_mul,
           nw1, nw2, noise1, noise2, s1_w, s1_b, s2_w, s2_b, s_mul):
    params = {"w0": w0, "w0_mul": w0_mul, "b0": b0, "w1": w1, "w1_mul": w1_mul,
              "b1": b1, "b_mul": b_mul, "nw1": nw1, "nw2": nw2,
              "noise1": noise1, "noise2": noise2, "s1_w": s1_w, "s1_b": s1_b,
              "s2_w": s2_w, "s2_b": s2_b, "s_mul": s_mul}
    return _forward(x_nchw, dlatents, params)
